# Initial kernel scaffold; baseline (speedup 1.0000x reference)
import jax
import jax.numpy as jnp
from jax.experimental import pallas as pl


def kernel(token_type, kos_token_index, identifier_index, encoded_identifiers, kos_table, kind_table, W_proj, b_proj):
    raise NotImplementedError("write your pallas kernel here")


# speedup vs baseline: 12.5526x; 12.5526x over previous
import functools

import jax
import jax.numpy as jnp
from jax import lax
from jax.experimental import pallas as pl
from jax.experimental.pallas import tpu as pltpu
from jax.experimental.pallas import tpu_sc as plsc

_IDENT_KIND = 4
_KIND_LO = 4
_KIND_HI = 7

_BLK = 128
_PAD = 512
_TC_ROWS = 2048


def _sc_gather_scatter(idx_kos, pos_kos, idx_id, pos_id, params,
                       kos_table, enc_table, n_pad_rows):
  info = plsc.get_sparse_core_info()
  nc, ns = info.num_cores, info.num_subcores
  mesh = plsc.VectorSubcoreMesh(core_axis_name="c", subcore_axis_name="s")

  @functools.partial(
      pl.kernel,
      mesh=mesh,
      compiler_params=pltpu.CompilerParams(use_tc_tiling_on_sc=False),
      out_type=jax.ShapeDtypeStruct((n_pad_rows, 64), jnp.float32),
      scratch_types=[
          pltpu.SMEM((16,), jnp.int32),
          pltpu.VMEM((16,), jnp.int32),
          pltpu.VMEM((_BLK,), jnp.int32),
          pltpu.VMEM((_BLK,), jnp.int32),
          pltpu.VMEM((_BLK, 64), jnp.float32),
      ],
  )
  def sc_kernel(idx_kos_h, pos_kos_h, idx_id_h, pos_id_h, params_h,
                kos_h, enc_h, base_h, pv, pvv, idxv, posv, rowsv):
    wid = lax.axis_index("s") * nc + lax.axis_index("c")
    pltpu.sync_copy(params_h, pvv)
    pvec = pvv[...]
    ck_kos = pvec[0]
    ck_id = pvec[1]

    def do_class(idx_h, pos_h, table_h, ck):
      nb = (ck + (_BLK - 1)) // _BLK
      start = wid * ck

      def blk(b, carry):
        c0 = pl.multiple_of(start + b * _BLK, 8)
        pltpu.sync_copy(idx_h.at[pl.ds(c0, _BLK)], idxv)
        pltpu.sync_copy(pos_h.at[pl.ds(c0, _BLK)], posv)
        pltpu.sync_copy(table_h.at[idxv], rowsv)
        pltpu.sync_copy(rowsv, base_h.at[posv])
        return carry

      lax.fori_loop(0, nb, blk, 0)

    do_class(idx_kos_h, pos_kos_h, kos_h, ck_kos)
    do_class(idx_id_h, pos_id_h, enc_h, ck_id)

  return sc_kernel(idx_kos, pos_kos, idx_id, pos_id, params,
                   kos_table, enc_table)


def _tc_project(tok3d, base, kind_table, w_proj, b_proj, n_rows):
  grid = n_rows // _TC_ROWS

  def tc_kernel(tok_ref, base_ref, kind_ref, w_ref, b_ref, out_ref):
    tokf = tok_ref[0, 0, :].astype(jnp.float32)
    tok_col = tokf.reshape(_TC_ROWS, 1)
    oh = (tok_col
          == lax.broadcasted_iota(jnp.int32, (_TC_ROWS, 16), 1
                                  ).astype(jnp.float32)
          ).astype(jnp.float32)
    kmat = jnp.dot(kind_ref[...], w_ref[:64, :],
                   preferred_element_type=jnp.float32)
    kind_part = jnp.dot(oh, kmat, preferred_element_type=jnp.float32)
    valid = (tok_col >= float(_KIND_LO)) & (tok_col <= float(_KIND_HI))
    base_m = jnp.where(jnp.broadcast_to(valid, (_TC_ROWS, 64)),
                       base_ref[...], 0.0)
    kos_part = jnp.dot(base_m, w_ref[64:, :],
                       preferred_element_type=jnp.float32)
    out_ref[...] = jnp.maximum(kind_part + kos_part + b_ref[0, :], 0.0)

  return pl.pallas_call(
      tc_kernel,
      grid=(grid,),
      in_specs=[
          pl.BlockSpec((1, 1, _TC_ROWS), lambda i: (i, 0, 0)),
          pl.BlockSpec((_TC_ROWS, 64), lambda i: (i, 0)),
          pl.BlockSpec((16, 64), lambda i: (0, 0)),
          pl.BlockSpec((128, 64), lambda i: (0, 0)),
          pl.BlockSpec((1, 64), lambda i: (0, 0)),
      ],
      out_specs=pl.BlockSpec((_TC_ROWS, 64), lambda i: (i, 0)),
      out_shape=jax.ShapeDtypeStruct((n_rows, 64), jnp.float32),
  )(tok3d, base, kind_table, w_proj, b_proj)


def kernel(token_type, kos_token_index, identifier_index, encoded_identifiers,
           kos_table, kind_table, W_proj, b_proj):
  bt, st = token_type.shape
  n = bt * st
  flat = token_type.reshape(-1)

  is_id = flat == _IDENT_KIND
  is_kos = (flat >= 5) & (flat <= 7)
  n_id = jnp.sum(is_id.astype(jnp.int32))
  n_kos = jnp.sum(is_kos.astype(jnp.int32))

  pos_id = jnp.nonzero(is_id, size=n, fill_value=n)[0].astype(jnp.int32)
  pos_kos = jnp.nonzero(is_kos, size=n, fill_value=n)[0].astype(jnp.int32)
  zpad = jnp.zeros((_PAD,), jnp.int32)
  npad = jnp.full((_PAD,), n, jnp.int32)
  pos_id = jnp.concatenate([pos_id, npad])
  pos_kos = jnp.concatenate([pos_kos, npad])
  idx_id = jnp.concatenate([identifier_index.astype(jnp.int32), zpad])
  idx_kos = jnp.concatenate([kos_token_index.astype(jnp.int32), zpad])

  nw = 32
  ck_kos = ((n_kos + nw - 1) // nw + 7) // 8 * 8
  ck_id = ((n_id + nw - 1) // nw + 7) // 8 * 8
  params = jnp.zeros((16,), jnp.int32)
  params = params.at[0].set(ck_kos).at[1].set(ck_id)

  n_pad_rows = _TC_ROWS * ((n + 8 + _TC_ROWS - 1) // _TC_ROWS)
  base = _sc_gather_scatter(idx_kos, pos_kos, idx_id, pos_id, params,
                            kos_table, encoded_identifiers, n_pad_rows)

  tok3d = flat.reshape(n // _TC_ROWS, 1, _TC_ROWS)
  out = _tc_project(tok3d, base, kind_table, W_proj,
                    b_proj.reshape(1, 64), n)
  return out.reshape(bt, st, 64)
